# baseline (speedup 1.0000x reference)
import jax
import jax.numpy as jnp
from jax.experimental import pallas as pl
from jax.experimental.pallas import tpu as pltpu

N = 10000
D = 128
F = 4
G = 64
H = 512
C = 10

BM = 400
M_BLOCKS = N // BM
P = 8


def _xw_kernel(x_ref, wg_ref, xw_ref):
    xw_ref[...] = jnp.dot(x_ref[...], wg_ref[...],
                          preferred_element_type=jnp.float32)


def _body_kernel(xw_ref, filt_ref, ids_ref, part_ref):
    h_blk = jnp.maximum(
        jnp.dot(filt_ref[...], xw_ref[...],
                preferred_element_type=jnp.float32), 0.0)
    gids = jax.lax.broadcasted_iota(jnp.int32, (BM, G), 1)
    oh = ids_ref[...] == gids
    cols = [jnp.max(jnp.where(oh, h_blk[:, f:f + 1], 0.0),
                    axis=0, keepdims=True) for f in range(F)]
    local = jnp.concatenate(
        cols + [jnp.zeros((P - F, G), jnp.float32)], axis=0)
    part_ref[...] = local


def _head_kernel(part_ref, wh_ref, bh_ref, wc_ref, bc_ref, out_ref):
    acc = part_ref[0:P, :]
    for i in range(1, M_BLOCKS):
        acc = jnp.maximum(acc, part_ref[i * P:(i + 1) * P, :])
    pooled_t = acc[0:F, :]
    z = jnp.maximum(
        jax.lax.dot_general(pooled_t, wh_ref[...],
                            (((0,), (0,)), ((), ())),
                            preferred_element_type=jnp.float32)
        + bh_ref[...], 0.0)
    logits = jnp.dot(z, wc_ref[...],
                     preferred_element_type=jnp.float32) + bc_ref[...]
    mx = jnp.max(logits, axis=-1, keepdims=True)
    e = jnp.exp(logits - mx)
    out_ref[...] = e / jnp.sum(e, axis=-1, keepdims=True)


@jax.jit
def _run(filtre, X, ids2, W_gcn, W_h, b_h, W_c, b_c):
    xw = pl.pallas_call(
        _xw_kernel,
        out_shape=jax.ShapeDtypeStruct((N, F), jnp.float32),
    )(X, W_gcn)

    partials = pl.pallas_call(
        _body_kernel,
        grid=(M_BLOCKS,),
        in_specs=[
            pl.BlockSpec((N, F), lambda m: (0, 0)),
            pl.BlockSpec((BM, N), lambda m: (m, 0)),
            pl.BlockSpec((BM, 1), lambda m: (m, 0)),
        ],
        out_specs=pl.BlockSpec((P, G), lambda m: (m, 0)),
        out_shape=jax.ShapeDtypeStruct((M_BLOCKS * P, G), jnp.float32),
        compiler_params=pltpu.CompilerParams(
            dimension_semantics=("parallel",)),
    )(xw, filtre, ids2)

    return pl.pallas_call(
        _head_kernel,
        out_shape=jax.ShapeDtypeStruct((G, C), jnp.float32),
    )(partials, W_h, b_h, W_c, b_c)


def kernel(filtre, X, node_indicator, W_gcn, W_h, b_h, W_c, b_c):
    ids2 = node_indicator.astype(jnp.int32).reshape(N, 1)
    return _run(filtre, X, ids2, W_gcn, W_h,
                b_h.reshape(1, H), W_c, b_c.reshape(1, C))


# speedup vs baseline: 1.0414x; 1.0414x over previous
import jax
import jax.numpy as jnp
from jax.experimental import pallas as pl
from jax.experimental.pallas import tpu as pltpu

N = 10000
D = 128
F = 4
G = 64
H = 512
C = 10

BM = 672
M_BLOCKS = (N + BM - 1) // BM


def _fused_kernel(x_ref, wg_ref, filt_ref, ids_ref, wh_ref, bh_ref,
                  wc_ref, bc_ref, out_ref, xw_ref, pooled_ref):
    m = pl.program_id(0)

    @pl.when(m == 0)
    def _init():
        xw_ref[...] = jnp.dot(x_ref[...], wg_ref[...],
                              preferred_element_type=jnp.float32)
        pooled_ref[...] = jnp.zeros_like(pooled_ref)

    h_blk = jnp.maximum(
        jnp.dot(filt_ref[...], xw_ref[...],
                preferred_element_type=jnp.float32), 0.0)

    gids = jax.lax.broadcasted_iota(jnp.int32, (BM, G), 1)
    rows = jax.lax.broadcasted_iota(jnp.int32, (BM, G), 0) + m * BM
    oh = (ids_ref[...] == gids) & (rows < N)
    cols = [jnp.max(jnp.where(oh, h_blk[:, f:f + 1], 0.0),
                    axis=0, keepdims=True) for f in range(F)]
    local = jnp.concatenate(cols, axis=0)
    pooled_ref[...] = jnp.maximum(pooled_ref[...], local)

    @pl.when(m == M_BLOCKS - 1)
    def _head():
        pooled_t = pooled_ref[...]
        z = jnp.maximum(
            jax.lax.dot_general(pooled_t, wh_ref[...],
                                (((0,), (0,)), ((), ())),
                                preferred_element_type=jnp.float32)
            + bh_ref[...], 0.0)
        logits = jnp.dot(z, wc_ref[...],
                         preferred_element_type=jnp.float32) + bc_ref[...]
        mx = jnp.max(logits, axis=-1, keepdims=True)
        e = jnp.exp(logits - mx)
        out_ref[...] = e / jnp.sum(e, axis=-1, keepdims=True)


@jax.jit
def _run(filtre, X, ids2, W_gcn, W_h, b_h, W_c, b_c):
    return pl.pallas_call(
        _fused_kernel,
        grid=(M_BLOCKS,),
        in_specs=[
            pl.BlockSpec((N, D), lambda m: (0, 0)),
            pl.BlockSpec((D, F), lambda m: (0, 0)),
            pl.BlockSpec((BM, N), lambda m: (m, 0)),
            pl.BlockSpec((BM, 1), lambda m: (m, 0)),
            pl.BlockSpec((F, H), lambda m: (0, 0)),
            pl.BlockSpec((1, H), lambda m: (0, 0)),
            pl.BlockSpec((H, C), lambda m: (0, 0)),
            pl.BlockSpec((1, C), lambda m: (0, 0)),
        ],
        out_specs=pl.BlockSpec((G, C), lambda m: (0, 0)),
        out_shape=jax.ShapeDtypeStruct((G, C), jnp.float32),
        compiler_params=pltpu.CompilerParams(
            vmem_limit_bytes=64 * 1024 * 1024),
        scratch_shapes=[
            pltpu.VMEM((N, F), jnp.float32),
            pltpu.VMEM((F, G), jnp.float32),
        ],
    )(X, W_gcn, filtre, ids2, W_h, b_h, W_c, b_c)


def kernel(filtre, X, node_indicator, W_gcn, W_h, b_h, W_c, b_c):
    ids2 = node_indicator.astype(jnp.int32).reshape(N, 1)
    return _run(filtre, X, ids2, W_gcn, W_h,
                b_h.reshape(1, H), W_c, b_c.reshape(1, C))
